# baseline (speedup 1.0000x reference)
import functools

import jax
import jax.numpy as jnp
from jax import lax
from jax.experimental import pallas as pl
from jax.experimental.pallas import tpu as pltpu
from jax.experimental.pallas import tpu_sc as plsc

_N = 10000
_E = 320000
_D = 128
_NC = 2
_NS = 16
_NW = _NC * _NS
_EPW = _E // _NW
_CH = 80
_NCH = _EPW // _CH
_ACH = 40
_ANCH = _EPW // _ACH
_RPT = 640
_NP = _RPT * _NS
_BLK = 1024
_GRID = _NP // _BLK


def _sc_mesh():
    return plsc.VectorSubcoreMesh(core_axis_name="c", subcore_axis_name="s")


@functools.partial(
    pl.kernel,
    out_type=jax.ShapeDtypeStruct((2 * _NP, _D), jnp.float32),
    mesh=_sc_mesh(),
    scratch_types=[
        pltpu.VMEM_SHARED((_NP, _D), jnp.float32),
        pltpu.VMEM((_NCH, _CH), jnp.int32),
        pltpu.VMEM((_CH, _D), jnp.float32),
    ],
)
def _sc_deg(dst3_hbm, zeros_hbm, ones_hbm, out_hbm, acc, didx, ones_v):
    c = lax.axis_index("c")
    s = lax.axis_index("s")
    wid = s * _NC + c
    pltpu.sync_copy(zeros_hbm.at[pl.ds(s * _RPT, _RPT)],
                    acc.at[pl.ds(s * _RPT, _RPT)])
    pltpu.sync_copy(ones_hbm, ones_v)
    pltpu.sync_copy(dst3_hbm.at[wid], didx)
    plsc.subcore_barrier()

    def step(j, carry):
        pltpu.sync_copy(ones_v, acc.at[didx.at[j]], add=True)
        return carry

    lax.fori_loop(0, _NCH, step, 0)
    plsc.subcore_barrier()
    pltpu.sync_copy(acc.at[pl.ds(s * _RPT, _RPT)],
                    out_hbm.at[pl.ds(c * _NP + s * _RPT, _RPT)])


@functools.partial(
    pl.kernel,
    out_type=jax.ShapeDtypeStruct((2 * _NP, _D), jnp.float32),
    mesh=_sc_mesh(),
    scratch_types=[
        pltpu.VMEM_SHARED((_NP, _D), jnp.float32),
        pltpu.VMEM((_EPW,), jnp.int32),
        pltpu.VMEM((_ACH,), jnp.int32),
        pltpu.VMEM((_ACH,), jnp.int32),
        pltpu.VMEM((_ACH,), jnp.int32),
        pltpu.VMEM((_ACH,), jnp.int32),
        pltpu.VMEM((_ACH, _D), jnp.float32),
        pltpu.VMEM((_ACH, _D), jnp.float32),
        pltpu.VMEM((_ACH, _D), jnp.float32),
        pltpu.VMEM((_ACH, _D), jnp.float32),
        pltpu.SemaphoreType.DMA,
        pltpu.SemaphoreType.DMA,
        pltpu.SemaphoreType.DMA,
        pltpu.SemaphoreType.DMA,
    ],
)
def _sc_agg(table_hbm, src_hbm, dst_hbm, zeros_hbm, out_hbm,
            acc, sidx, di0, di1, di2, di3, ro0, ro1, ro2, ro3,
            se0, se1, se2, se3):
    c = lax.axis_index("c")
    s = lax.axis_index("s")
    wid = s * _NC + c
    ebase = wid * _EPW
    pltpu.sync_copy(zeros_hbm.at[pl.ds(s * _RPT, _RPT)],
                    acc.at[pl.ds(s * _RPT, _RPT)])
    pltpu.sync_copy(src_hbm.at[pl.ds(ebase, _EPW)], sidx)
    plsc.subcore_barrier()

    slots = ((di0, ro0, se0), (di1, ro1, se1), (di2, ro2, se2), (di3, ro3, se3))

    def sch(j):
        return sidx.at[pl.ds(pl.multiple_of(j * _ACH, _ACH), _ACH)]

    def start(j, b):
        di, ro, se = slots[b]
        pltpu.async_copy(dst_hbm.at[pl.ds(ebase + j * _ACH, _ACH)], di, se)
        pltpu.async_copy(table_hbm.at[sch(j)], ro, se)

    def finish(j, b):
        di, ro, se = slots[b]
        pltpu.make_async_copy(dst_hbm.at[pl.ds(ebase, _ACH)], di, se).wait()
        pltpu.make_async_copy(table_hbm.at[sch(j)], ro, se).wait()
        pltpu.sync_copy(ro, acc.at[di], add=True)

    start(0, 0)
    start(1, 1)

    def quad(i, carry):
        j = 4 * i
        finish(j, 0)
        start(j + 2, 2)
        finish(j + 1, 1)
        start(j + 3, 3)
        finish(j + 2, 2)
        start(j + 4, 0)
        finish(j + 3, 3)
        start(j + 5, 1)
        return carry

    lax.fori_loop(0, (_ANCH - 2) // 4, quad, 0)
    finish(_ANCH - 2, 0)
    finish(_ANCH - 1, 1)
    plsc.subcore_barrier()
    pltpu.sync_copy(acc.at[pl.ds(s * _RPT, _RPT)],
                    out_hbm.at[pl.ds(c * _NP + s * _RPT, _RPT)])


def _tc_first_body(deg0_ref, deg1_ref, x_ref, w_ref, dis_ref, hwp_ref):
    p0 = deg0_ref[:, 0:1]
    p1 = deg1_ref[:, 0:1]
    dis = lax.rsqrt(1.0 + p0 + p1)
    dis_b = jnp.broadcast_to(dis, (_BLK, _D))
    hw = jnp.dot(x_ref[...], w_ref[...], preferred_element_type=jnp.float32)
    dis_ref[...] = dis_b
    hwp_ref[...] = hw * dis_b


def _tc_first(deg, x, W1):
    return pl.pallas_call(
        _tc_first_body,
        grid=(_GRID,),
        in_specs=[
            pl.BlockSpec((_BLK, _D), lambda j: (j, 0)),
            pl.BlockSpec((_BLK, _D), lambda j: (j + _GRID, 0)),
            pl.BlockSpec((_BLK, _D), lambda j: (j, 0)),
            pl.BlockSpec((_D, _D), lambda j: (0, 0)),
        ],
        out_specs=[
            pl.BlockSpec((_BLK, _D), lambda j: (j, 0)),
            pl.BlockSpec((_BLK, _D), lambda j: (j, 0)),
        ],
        out_shape=[
            jax.ShapeDtypeStruct((_NP, _D), jnp.float32),
            jax.ShapeDtypeStruct((_NP, _D), jnp.float32),
        ],
    )(deg, deg, x, W1)


def _tc_mid_body(s0_ref, s1_ref, hwp_ref, dis_ref, b_ref, w_ref, out_ref):
    h = dis_ref[...] * (s0_ref[...] + s1_ref[...] + hwp_ref[...]) + b_ref[...]
    h = jnp.maximum(h, 0.0)
    out_ref[...] = (
        jnp.dot(h, w_ref[...], preferred_element_type=jnp.float32)
        * dis_ref[...]
    )


def _tc_mid(S, hwp, dis, b, W):
    return pl.pallas_call(
        _tc_mid_body,
        grid=(_GRID,),
        in_specs=[
            pl.BlockSpec((_BLK, _D), lambda j: (j, 0)),
            pl.BlockSpec((_BLK, _D), lambda j: (j + _GRID, 0)),
            pl.BlockSpec((_BLK, _D), lambda j: (j, 0)),
            pl.BlockSpec((_BLK, _D), lambda j: (j, 0)),
            pl.BlockSpec((1, _D), lambda j: (0, 0)),
            pl.BlockSpec((_D, _D), lambda j: (0, 0)),
        ],
        out_specs=pl.BlockSpec((_BLK, _D), lambda j: (j, 0)),
        out_shape=jax.ShapeDtypeStruct((_NP, _D), jnp.float32),
    )(S, S, hwp, dis, b, W)


def _tc_last_body(s0_ref, s1_ref, hwp_ref, dis_ref, b_ref, wo_ref, bo_ref,
                  out_ref):
    h = dis_ref[...] * (s0_ref[...] + s1_ref[...] + hwp_ref[...]) + b_ref[...]
    h = jnp.maximum(h, 0.0)
    z = jnp.dot(h, wo_ref[...], preferred_element_type=jnp.float32) + bo_ref[...]
    out_ref[...] = jax.nn.sigmoid(z)


def _tc_last(S, hwp, dis, b3, Wo, bo):
    return pl.pallas_call(
        _tc_last_body,
        grid=(_GRID,),
        in_specs=[
            pl.BlockSpec((_BLK, _D), lambda j: (j, 0)),
            pl.BlockSpec((_BLK, _D), lambda j: (j + _GRID, 0)),
            pl.BlockSpec((_BLK, _D), lambda j: (j, 0)),
            pl.BlockSpec((_BLK, _D), lambda j: (j, 0)),
            pl.BlockSpec((1, _D), lambda j: (0, 0)),
            pl.BlockSpec((_D, 1), lambda j: (0, 0)),
            pl.BlockSpec((1, 1), lambda j: (0, 0)),
        ],
        out_specs=pl.BlockSpec((_BLK, 1), lambda j: (j, 0)),
        out_shape=jax.ShapeDtypeStruct((_N, 1), jnp.float32),
    )(S, S, hwp, dis, b3, Wo, bo)


def kernel(x, edge_index, W1, b1, W2, b2, W3, b3, Wo, bo):
    src1 = edge_index[0]
    dst2 = edge_index[1].reshape(_NW, _NCH, _CH)
    zeros = jnp.zeros((_NP, _D), jnp.float32)
    ones = jnp.ones((_CH, _D), jnp.float32)

    deg = _sc_deg(dst2, zeros, ones)
    dis, hw1p = _tc_first(deg, x, W1)
    dst1 = edge_index[1]
    S1 = _sc_agg(hw1p, src1, dst1, zeros)
    hw2p = _tc_mid(S1, hw1p, dis, b1.reshape(1, _D), W2)
    S2 = _sc_agg(hw2p, src1, dst1, zeros)
    hw3p = _tc_mid(S2, hw2p, dis, b2.reshape(1, _D), W3)
    S3 = _sc_agg(hw3p, src1, dst1, zeros)
    return _tc_last(S3, hw3p, dis, b3.reshape(1, _D), Wo, bo.reshape(1, 1))


# speedup vs baseline: 1.2354x; 1.2354x over previous
import functools

import jax
import jax.numpy as jnp
from jax import lax
from jax.experimental import pallas as pl
from jax.experimental.pallas import tpu as pltpu
from jax.experimental.pallas import tpu_sc as plsc

_N = 10000
_E = 320000
_D = 128
_NC = 2
_NS = 16
_NW = _NC * _NS
_EPW = _E // _NW
_CH = 80
_NCH = _EPW // _CH
_ACH = 80
_ANCH = _EPW // _ACH
_RPT = 640
_NP = _RPT * _NS
_BLK = 1024
_GRID = _NP // _BLK


def _sc_mesh():
    return plsc.VectorSubcoreMesh(core_axis_name="c", subcore_axis_name="s")


@functools.partial(
    pl.kernel,
    out_type=jax.ShapeDtypeStruct((2 * _NP, _D), jnp.float32),
    mesh=_sc_mesh(),
    scratch_types=[
        pltpu.VMEM_SHARED((_NP, _D), jnp.float32),
        pltpu.VMEM((_NCH, _CH), jnp.int32),
        pltpu.VMEM((_CH, _D), jnp.float32),
    ],
)
def _sc_deg(dst3_hbm, zeros_hbm, ones_hbm, out_hbm, acc, didx, ones_v):
    c = lax.axis_index("c")
    s = lax.axis_index("s")
    wid = s * _NC + c
    pltpu.sync_copy(zeros_hbm.at[pl.ds(s * _RPT, _RPT)],
                    acc.at[pl.ds(s * _RPT, _RPT)])
    pltpu.sync_copy(ones_hbm, ones_v)
    pltpu.sync_copy(dst3_hbm.at[wid], didx)
    plsc.subcore_barrier()

    def step(j, carry):
        pltpu.sync_copy(ones_v, acc.at[didx.at[j]], add=True)
        return carry

    lax.fori_loop(0, _NCH, step, 0)
    plsc.subcore_barrier()
    pltpu.sync_copy(acc.at[pl.ds(s * _RPT, _RPT)],
                    out_hbm.at[pl.ds(c * _NP + s * _RPT, _RPT)])


@functools.partial(
    pl.kernel,
    out_type=jax.ShapeDtypeStruct((2 * _NP, _D), jnp.float32),
    mesh=_sc_mesh(),
    scratch_types=[
        pltpu.VMEM_SHARED((_NP, _D), jnp.float32),
        pltpu.VMEM((_EPW,), jnp.int32),
        pltpu.VMEM((_ACH,), jnp.int32),
        pltpu.VMEM((_ACH,), jnp.int32),
        pltpu.VMEM((_ACH,), jnp.int32),
        pltpu.VMEM((_ACH, _D), jnp.float32),
        pltpu.VMEM((_ACH, _D), jnp.float32),
        pltpu.VMEM((_ACH, _D), jnp.float32),
        pltpu.SemaphoreType.DMA,
        pltpu.SemaphoreType.DMA,
        pltpu.SemaphoreType.DMA,
    ],
)
def _sc_agg(table_hbm, src_hbm, dst_hbm, zeros_hbm, out_hbm,
            acc, sidx, di0, di1, di2, ro0, ro1, ro2,
            se0, se1, se2):
    c = lax.axis_index("c")
    s = lax.axis_index("s")
    wid = s * _NC + c
    ebase = wid * _EPW
    pltpu.sync_copy(zeros_hbm.at[pl.ds(s * _RPT, _RPT)],
                    acc.at[pl.ds(s * _RPT, _RPT)])
    pltpu.sync_copy(src_hbm.at[pl.ds(ebase, _EPW)], sidx)
    plsc.subcore_barrier()

    slots = ((di0, ro0, se0), (di1, ro1, se1), (di2, ro2, se2))

    def sch(j):
        return sidx.at[pl.ds(pl.multiple_of(j * _ACH, _ACH), _ACH)]

    def start(j, b):
        di, ro, se = slots[b]
        pltpu.async_copy(dst_hbm.at[pl.ds(ebase + j * _ACH, _ACH)], di, se)
        pltpu.async_copy(table_hbm.at[sch(j)], ro, se)

    def finish(j, b):
        di, ro, se = slots[b]
        pltpu.make_async_copy(dst_hbm.at[pl.ds(ebase, _ACH)], di, se).wait()
        pltpu.make_async_copy(table_hbm.at[sch(j)], ro, se).wait()
        pltpu.sync_copy(ro, acc.at[di], add=True)

    start(0, 0)
    start(1, 1)

    def trip(i, carry):
        j = 3 * i
        finish(j, 0)
        start(j + 2, 2)
        finish(j + 1, 1)
        start(j + 3, 0)
        finish(j + 2, 2)
        start(j + 4, 1)
        return carry

    lax.fori_loop(0, (_ANCH - 2) // 3, trip, 0)
    finish(_ANCH - 2, 0)
    finish(_ANCH - 1, 1)
    plsc.subcore_barrier()
    pltpu.sync_copy(acc.at[pl.ds(s * _RPT, _RPT)],
                    out_hbm.at[pl.ds(c * _NP + s * _RPT, _RPT)])


def _tc_first_body(deg0_ref, deg1_ref, x_ref, w_ref, dis_ref, hwp_ref):
    p0 = deg0_ref[:, 0:1]
    p1 = deg1_ref[:, 0:1]
    dis = lax.rsqrt(1.0 + p0 + p1)
    dis_b = jnp.broadcast_to(dis, (_BLK, _D))
    hw = jnp.dot(x_ref[...], w_ref[...], preferred_element_type=jnp.float32)
    dis_ref[...] = dis_b
    hwp_ref[...] = hw * dis_b


def _tc_first(deg, x, W1):
    return pl.pallas_call(
        _tc_first_body,
        grid=(_GRID,),
        in_specs=[
            pl.BlockSpec((_BLK, _D), lambda j: (j, 0)),
            pl.BlockSpec((_BLK, _D), lambda j: (j + _GRID, 0)),
            pl.BlockSpec((_BLK, _D), lambda j: (j, 0)),
            pl.BlockSpec((_D, _D), lambda j: (0, 0)),
        ],
        out_specs=[
            pl.BlockSpec((_BLK, _D), lambda j: (j, 0)),
            pl.BlockSpec((_BLK, _D), lambda j: (j, 0)),
        ],
        out_shape=[
            jax.ShapeDtypeStruct((_NP, _D), jnp.float32),
            jax.ShapeDtypeStruct((_NP, _D), jnp.float32),
        ],
    )(deg, deg, x, W1)


def _tc_mid_body(s0_ref, s1_ref, hwp_ref, dis_ref, b_ref, w_ref, out_ref):
    h = dis_ref[...] * (s0_ref[...] + s1_ref[...] + hwp_ref[...]) + b_ref[...]
    h = jnp.maximum(h, 0.0)
    out_ref[...] = (
        jnp.dot(h, w_ref[...], preferred_element_type=jnp.float32)
        * dis_ref[...]
    )


def _tc_mid(S, hwp, dis, b, W):
    return pl.pallas_call(
        _tc_mid_body,
        grid=(_GRID,),
        in_specs=[
            pl.BlockSpec((_BLK, _D), lambda j: (j, 0)),
            pl.BlockSpec((_BLK, _D), lambda j: (j + _GRID, 0)),
            pl.BlockSpec((_BLK, _D), lambda j: (j, 0)),
            pl.BlockSpec((_BLK, _D), lambda j: (j, 0)),
            pl.BlockSpec((1, _D), lambda j: (0, 0)),
            pl.BlockSpec((_D, _D), lambda j: (0, 0)),
        ],
        out_specs=pl.BlockSpec((_BLK, _D), lambda j: (j, 0)),
        out_shape=jax.ShapeDtypeStruct((_NP, _D), jnp.float32),
    )(S, S, hwp, dis, b, W)


def _tc_last_body(s0_ref, s1_ref, hwp_ref, dis_ref, b_ref, wo_ref, bo_ref,
                  out_ref):
    h = dis_ref[...] * (s0_ref[...] + s1_ref[...] + hwp_ref[...]) + b_ref[...]
    h = jnp.maximum(h, 0.0)
    z = jnp.dot(h, wo_ref[...], preferred_element_type=jnp.float32) + bo_ref[...]
    out_ref[...] = jax.nn.sigmoid(z)


def _tc_last(S, hwp, dis, b3, Wo, bo):
    return pl.pallas_call(
        _tc_last_body,
        grid=(_GRID,),
        in_specs=[
            pl.BlockSpec((_BLK, _D), lambda j: (j, 0)),
            pl.BlockSpec((_BLK, _D), lambda j: (j + _GRID, 0)),
            pl.BlockSpec((_BLK, _D), lambda j: (j, 0)),
            pl.BlockSpec((_BLK, _D), lambda j: (j, 0)),
            pl.BlockSpec((1, _D), lambda j: (0, 0)),
            pl.BlockSpec((_D, 1), lambda j: (0, 0)),
            pl.BlockSpec((1, 1), lambda j: (0, 0)),
        ],
        out_specs=pl.BlockSpec((_BLK, 1), lambda j: (j, 0)),
        out_shape=jax.ShapeDtypeStruct((_N, 1), jnp.float32),
    )(S, S, hwp, dis, b3, Wo, bo)


def kernel(x, edge_index, W1, b1, W2, b2, W3, b3, Wo, bo):
    src1 = edge_index[0]
    dst2 = edge_index[1].reshape(_NW, _NCH, _CH)
    zeros = jnp.zeros((_NP, _D), jnp.float32)
    ones = jnp.ones((_CH, _D), jnp.float32)

    deg = _sc_deg(dst2, zeros, ones)
    dis, hw1p = _tc_first(deg, x, W1)
    dst1 = edge_index[1]
    S1 = _sc_agg(hw1p, src1, dst1, zeros)
    hw2p = _tc_mid(S1, hw1p, dis, b1.reshape(1, _D), W2)
    S2 = _sc_agg(hw2p, src1, dst1, zeros)
    hw3p = _tc_mid(S2, hw2p, dis, b2.reshape(1, _D), W3)
    S3 = _sc_agg(hw3p, src1, dst1, zeros)
    return _tc_last(S3, hw3p, dis, b3.reshape(1, _D), Wo, bo.reshape(1, 1))


# speedup vs baseline: 1.4201x; 1.1494x over previous
import functools

import jax
import jax.numpy as jnp
from jax import lax
from jax.experimental import pallas as pl
from jax.experimental.pallas import tpu as pltpu
from jax.experimental.pallas import tpu_sc as plsc

_N = 10000
_E = 320000
_D = 128
_NC = 2
_NS = 16
_NW = _NC * _NS
_EPW = _E // _NW
_CH = 80
_NCH = _EPW // _CH
_ACH = 80
_ANCH = _EPW // _ACH
_RPT = 640
_NP = _RPT * _NS
_BLK = 1024
_GRID = _NP // _BLK


def _sc_mesh():
    return plsc.VectorSubcoreMesh(core_axis_name="c", subcore_axis_name="s")


@functools.partial(
    pl.kernel,
    out_type=jax.ShapeDtypeStruct((2 * _NP, _D), jnp.float32),
    mesh=_sc_mesh(),
    scratch_types=[
        pltpu.VMEM_SHARED((_NP, _D), jnp.float32),
        pltpu.VMEM((_NCH, _CH), jnp.int32),
        pltpu.VMEM((_CH, _D), jnp.float32),
    ],
)
def _sc_deg(dst3_hbm, zeros_hbm, ones_hbm, out_hbm, acc, didx, ones_v):
    c = lax.axis_index("c")
    s = lax.axis_index("s")
    wid = s * _NC + c
    pltpu.sync_copy(zeros_hbm.at[pl.ds(s * _RPT, _RPT)],
                    acc.at[pl.ds(s * _RPT, _RPT)])
    pltpu.sync_copy(ones_hbm, ones_v)
    pltpu.sync_copy(dst3_hbm.at[wid], didx)
    plsc.subcore_barrier()

    def step(j, carry):
        pltpu.sync_copy(ones_v, acc.at[didx.at[j]], add=True)
        return carry

    lax.fori_loop(0, _NCH, step, 0)
    plsc.subcore_barrier()
    pltpu.sync_copy(acc.at[pl.ds(s * _RPT, _RPT)],
                    out_hbm.at[pl.ds(c * _NP + s * _RPT, _RPT)])


@functools.partial(
    pl.kernel,
    out_type=jax.ShapeDtypeStruct((2 * _NP, _D), jnp.float32),
    mesh=_sc_mesh(),
    scratch_types=[
        pltpu.VMEM_SHARED((_NP, _D), jnp.float32),
        pltpu.VMEM((_ACH,), jnp.int32),
        pltpu.VMEM((_ACH,), jnp.int32),
        pltpu.VMEM((_ACH,), jnp.int32),
        pltpu.VMEM((_ACH,), jnp.int32),
        pltpu.VMEM((_ACH,), jnp.int32),
        pltpu.VMEM((_ACH,), jnp.int32),
        pltpu.VMEM((_ACH,), jnp.int32),
        pltpu.VMEM((_ACH,), jnp.int32),
        pltpu.VMEM((_ACH, _D), jnp.float32),
        pltpu.VMEM((_ACH, _D), jnp.float32),
        pltpu.VMEM((_ACH, _D), jnp.float32),
        pltpu.VMEM((_ACH, _D), jnp.float32),
        pltpu.SemaphoreType.DMA,
        pltpu.SemaphoreType.DMA,
        pltpu.SemaphoreType.DMA,
        pltpu.SemaphoreType.DMA,
        pltpu.SemaphoreType.DMA,
        pltpu.SemaphoreType.DMA,
        pltpu.SemaphoreType.DMA,
        pltpu.SemaphoreType.DMA,
    ],
)
def _sc_agg(table_hbm, src_hbm, dst_hbm, zeros_hbm, out_hbm,
            acc, si0, si1, si2, si3, di0, di1, di2, di3,
            ro0, ro1, ro2, ro3,
            sa0, sa1, sa2, sa3, sg0, sg1, sg2, sg3):
    c = lax.axis_index("c")
    s = lax.axis_index("s")
    wid = s * _NC + c
    ebase = wid * _EPW
    pltpu.sync_copy(zeros_hbm.at[pl.ds(s * _RPT, _RPT)],
                    acc.at[pl.ds(s * _RPT, _RPT)])
    plsc.subcore_barrier()

    slots = ((si0, di0, ro0, sa0, sg0), (si1, di1, ro1, sa1, sg1),
             (si2, di2, ro2, sa2, sg2), (si3, di3, ro3, sa3, sg3))

    def start(j, b):
        si, di, ro, sa, sg = slots[b]
        pltpu.async_copy(src_hbm.at[pl.ds(ebase + j * _ACH, _ACH)], si, sa)
        pltpu.async_copy(dst_hbm.at[pl.ds(ebase + j * _ACH, _ACH)], di, sa)

    def arm(j, b):
        si, di, ro, sa, sg = slots[b]
        pltpu.make_async_copy(src_hbm.at[pl.ds(ebase, _ACH)], si, sa).wait()
        pltpu.make_async_copy(dst_hbm.at[pl.ds(ebase, _ACH)], di, sa).wait()
        pltpu.async_copy(table_hbm.at[si], ro, sg)

    def finish(j, b):
        si, di, ro, sa, sg = slots[b]
        pltpu.make_async_copy(table_hbm.at[si], ro, sg).wait()
        pltpu.sync_copy(ro, acc.at[di], add=True)

    start(0, 0)
    start(1, 1)
    start(2, 2)
    start(3, 3)
    arm(0, 0)
    arm(1, 1)
    arm(2, 2)

    def quad(i, carry):
        j = 4 * i
        finish(j, 0)
        start(j + 4, 0)
        arm(j + 3, 3)
        finish(j + 1, 1)
        start(j + 5, 1)
        arm(j + 4, 0)
        finish(j + 2, 2)
        start(j + 6, 2)
        arm(j + 5, 1)
        finish(j + 3, 3)
        start(j + 7, 3)
        arm(j + 6, 2)
        return carry

    lax.fori_loop(0, (_ANCH - 5) // 4, quad, 0)
    jj = _ANCH - 5
    finish(jj, jj % 4)
    start(jj + 4, (jj + 4) % 4)
    arm(jj + 3, (jj + 3) % 4)
    finish(jj + 1, (jj + 1) % 4)
    arm(jj + 4, (jj + 4) % 4)
    finish(jj + 2, (jj + 2) % 4)
    finish(jj + 3, (jj + 3) % 4)
    finish(jj + 4, (jj + 4) % 4)
    plsc.subcore_barrier()
    pltpu.sync_copy(acc.at[pl.ds(s * _RPT, _RPT)],
                    out_hbm.at[pl.ds(c * _NP + s * _RPT, _RPT)])


def _tc_first_body(deg0_ref, deg1_ref, x_ref, w_ref, dis_ref, hwp_ref):
    p0 = deg0_ref[:, 0:1]
    p1 = deg1_ref[:, 0:1]
    dis = lax.rsqrt(1.0 + p0 + p1)
    dis_b = jnp.broadcast_to(dis, (_BLK, _D))
    hw = jnp.dot(x_ref[...], w_ref[...], preferred_element_type=jnp.float32)
    dis_ref[...] = dis_b
    hwp_ref[...] = hw * dis_b


def _tc_first(deg, x, W1):
    return pl.pallas_call(
        _tc_first_body,
        grid=(_GRID,),
        in_specs=[
            pl.BlockSpec((_BLK, _D), lambda j: (j, 0)),
            pl.BlockSpec((_BLK, _D), lambda j: (j + _GRID, 0)),
            pl.BlockSpec((_BLK, _D), lambda j: (j, 0)),
            pl.BlockSpec((_D, _D), lambda j: (0, 0)),
        ],
        out_specs=[
            pl.BlockSpec((_BLK, _D), lambda j: (j, 0)),
            pl.BlockSpec((_BLK, _D), lambda j: (j, 0)),
        ],
        out_shape=[
            jax.ShapeDtypeStruct((_NP, _D), jnp.float32),
            jax.ShapeDtypeStruct((_NP, _D), jnp.float32),
        ],
    )(deg, deg, x, W1)


def _tc_mid_body(s0_ref, s1_ref, hwp_ref, dis_ref, b_ref, w_ref, out_ref):
    h = dis_ref[...] * (s0_ref[...] + s1_ref[...] + hwp_ref[...]) + b_ref[...]
    h = jnp.maximum(h, 0.0)
    out_ref[...] = (
        jnp.dot(h, w_ref[...], preferred_element_type=jnp.float32)
        * dis_ref[...]
    )


def _tc_mid(S, hwp, dis, b, W):
    return pl.pallas_call(
        _tc_mid_body,
        grid=(_GRID,),
        in_specs=[
            pl.BlockSpec((_BLK, _D), lambda j: (j, 0)),
            pl.BlockSpec((_BLK, _D), lambda j: (j + _GRID, 0)),
            pl.BlockSpec((_BLK, _D), lambda j: (j, 0)),
            pl.BlockSpec((_BLK, _D), lambda j: (j, 0)),
            pl.BlockSpec((1, _D), lambda j: (0, 0)),
            pl.BlockSpec((_D, _D), lambda j: (0, 0)),
        ],
        out_specs=pl.BlockSpec((_BLK, _D), lambda j: (j, 0)),
        out_shape=jax.ShapeDtypeStruct((_NP, _D), jnp.float32),
    )(S, S, hwp, dis, b, W)


def _tc_last_body(s0_ref, s1_ref, hwp_ref, dis_ref, b_ref, wo_ref, bo_ref,
                  out_ref):
    h = dis_ref[...] * (s0_ref[...] + s1_ref[...] + hwp_ref[...]) + b_ref[...]
    h = jnp.maximum(h, 0.0)
    z = jnp.dot(h, wo_ref[...], preferred_element_type=jnp.float32) + bo_ref[...]
    out_ref[...] = jax.nn.sigmoid(z)


def _tc_last(S, hwp, dis, b3, Wo, bo):
    return pl.pallas_call(
        _tc_last_body,
        grid=(_GRID,),
        in_specs=[
            pl.BlockSpec((_BLK, _D), lambda j: (j, 0)),
            pl.BlockSpec((_BLK, _D), lambda j: (j + _GRID, 0)),
            pl.BlockSpec((_BLK, _D), lambda j: (j, 0)),
            pl.BlockSpec((_BLK, _D), lambda j: (j, 0)),
            pl.BlockSpec((1, _D), lambda j: (0, 0)),
            pl.BlockSpec((_D, 1), lambda j: (0, 0)),
            pl.BlockSpec((1, 1), lambda j: (0, 0)),
        ],
        out_specs=pl.BlockSpec((_BLK, 1), lambda j: (j, 0)),
        out_shape=jax.ShapeDtypeStruct((_N, 1), jnp.float32),
    )(S, S, hwp, dis, b3, Wo, bo)


def kernel(x, edge_index, W1, b1, W2, b2, W3, b3, Wo, bo):
    src1 = edge_index[0]
    dst2 = edge_index[1].reshape(_NW, _NCH, _CH)
    zeros = jnp.zeros((_NP, _D), jnp.float32)
    ones = jnp.ones((_CH, _D), jnp.float32)

    deg = _sc_deg(dst2, zeros, ones)
    dis, hw1p = _tc_first(deg, x, W1)
    dst1 = edge_index[1]
    S1 = _sc_agg(hw1p, src1, dst1, zeros)
    hw2p = _tc_mid(S1, hw1p, dis, b1.reshape(1, _D), W2)
    S2 = _sc_agg(hw2p, src1, dst1, zeros)
    hw3p = _tc_mid(S2, hw2p, dis, b2.reshape(1, _D), W3)
    S3 = _sc_agg(hw3p, src1, dst1, zeros)
    return _tc_last(S3, hw3p, dis, b3.reshape(1, _D), Wo, bo.reshape(1, 1))


# speedup vs baseline: 1.4250x; 1.0035x over previous
import functools

import jax
import jax.numpy as jnp
from jax import lax
from jax.experimental import pallas as pl
from jax.experimental.pallas import tpu as pltpu
from jax.experimental.pallas import tpu_sc as plsc

_N = 10000
_E = 320000
_D = 128
_NC = 2
_NS = 16
_NW = _NC * _NS
_EPW = _E // _NW
_CH = 80
_NCH = _EPW // _CH
_ACH = 80
_ANCH = _EPW // _ACH
_RPT = 640
_NP = _RPT * _NS
_BLK = 1024
_GRID = _NP // _BLK


def _sc_mesh():
    return plsc.VectorSubcoreMesh(core_axis_name="c", subcore_axis_name="s")


@functools.partial(
    pl.kernel,
    out_type=jax.ShapeDtypeStruct((2 * _NP, _D), jnp.float32),
    mesh=_sc_mesh(),
    scratch_types=[
        pltpu.VMEM_SHARED((_NP, _D), jnp.float32),
        pltpu.VMEM((_NCH, _CH), jnp.int32),
        pltpu.VMEM((_CH, _D), jnp.float32),
        pltpu.SemaphoreType.DMA,
        pltpu.SemaphoreType.DMA,
    ],
)
def _sc_deg(dst3_hbm, zeros_hbm, ones_hbm, out_hbm, acc, didx, ones_v,
            sm0, sm1):
    c = lax.axis_index("c")
    s = lax.axis_index("s")
    wid = s * _NC + c
    pltpu.sync_copy(zeros_hbm.at[pl.ds(s * _RPT, _RPT)],
                    acc.at[pl.ds(s * _RPT, _RPT)])
    pltpu.sync_copy(ones_hbm, ones_v)
    pltpu.sync_copy(dst3_hbm.at[wid], didx)
    plsc.subcore_barrier()

    def issue(j, sm):
        pltpu.async_copy(ones_v, acc.at[didx.at[j]], sm, add=True)

    def drain(j, sm):
        pltpu.make_async_copy(ones_v, acc.at[didx.at[j]], sm).wait()

    issue(0, sm0)

    def step2(i, carry):
        j = 2 * i
        issue(j + 1, sm1)
        drain(j, sm0)

        @pl.when(j + 2 < _NCH)
        def _():
            issue(j + 2, sm0)

        drain(j + 1, sm1)
        return carry

    lax.fori_loop(0, (_NCH - 1) // 2, step2, 0)
    drain(_NCH - 1, sm0)
    plsc.subcore_barrier()
    pltpu.sync_copy(acc.at[pl.ds(s * _RPT, _RPT)],
                    out_hbm.at[pl.ds(c * _NP + s * _RPT, _RPT)])


@functools.partial(
    pl.kernel,
    out_type=jax.ShapeDtypeStruct((2 * _NP, _D), jnp.float32),
    mesh=_sc_mesh(),
    scratch_types=[
        pltpu.VMEM_SHARED((_NP, _D), jnp.float32),
        pltpu.VMEM((_ACH,), jnp.int32),
        pltpu.VMEM((_ACH,), jnp.int32),
        pltpu.VMEM((_ACH,), jnp.int32),
        pltpu.VMEM((_ACH,), jnp.int32),
        pltpu.VMEM((_ACH,), jnp.int32),
        pltpu.VMEM((_ACH,), jnp.int32),
        pltpu.VMEM((_ACH,), jnp.int32),
        pltpu.VMEM((_ACH,), jnp.int32),
        pltpu.VMEM((_ACH, _D), jnp.float32),
        pltpu.VMEM((_ACH, _D), jnp.float32),
        pltpu.VMEM((_ACH, _D), jnp.float32),
        pltpu.VMEM((_ACH, _D), jnp.float32),
        pltpu.SemaphoreType.DMA,
        pltpu.SemaphoreType.DMA,
        pltpu.SemaphoreType.DMA,
        pltpu.SemaphoreType.DMA,
        pltpu.SemaphoreType.DMA,
        pltpu.SemaphoreType.DMA,
        pltpu.SemaphoreType.DMA,
        pltpu.SemaphoreType.DMA,
    ],
)
def _sc_agg(table_hbm, src_hbm, dst_hbm, zeros_hbm, out_hbm,
            acc, si0, si1, si2, si3, di0, di1, di2, di3,
            ro0, ro1, ro2, ro3,
            sa0, sa1, sa2, sa3, sg0, sg1, sg2, sg3):
    c = lax.axis_index("c")
    s = lax.axis_index("s")
    wid = s * _NC + c
    ebase = wid * _EPW
    pltpu.sync_copy(zeros_hbm.at[pl.ds(s * _RPT, _RPT)],
                    acc.at[pl.ds(s * _RPT, _RPT)])
    plsc.subcore_barrier()

    slots = ((si0, di0, ro0, sa0, sg0), (si1, di1, ro1, sa1, sg1),
             (si2, di2, ro2, sa2, sg2), (si3, di3, ro3, sa3, sg3))

    def start(j, b):
        si, di, ro, sa, sg = slots[b]
        pltpu.async_copy(src_hbm.at[pl.ds(ebase + j * _ACH, _ACH)], si, sa)
        pltpu.async_copy(dst_hbm.at[pl.ds(ebase + j * _ACH, _ACH)], di, sa)

    def arm(j, b):
        si, di, ro, sa, sg = slots[b]
        pltpu.make_async_copy(src_hbm.at[pl.ds(ebase, _ACH)], si, sa).wait()
        pltpu.make_async_copy(dst_hbm.at[pl.ds(ebase, _ACH)], di, sa).wait()
        pltpu.async_copy(table_hbm.at[si], ro, sg)

    def finish(j, b):
        si, di, ro, sa, sg = slots[b]
        pltpu.make_async_copy(table_hbm.at[si], ro, sg).wait()
        pltpu.sync_copy(ro, acc.at[di], add=True)

    start(0, 0)
    start(1, 1)
    start(2, 2)
    start(3, 3)
    arm(0, 0)
    arm(1, 1)
    arm(2, 2)

    def quad(i, carry):
        j = 4 * i
        finish(j, 0)
        start(j + 4, 0)
        arm(j + 3, 3)
        finish(j + 1, 1)
        start(j + 5, 1)
        arm(j + 4, 0)
        finish(j + 2, 2)
        start(j + 6, 2)
        arm(j + 5, 1)
        finish(j + 3, 3)
        start(j + 7, 3)
        arm(j + 6, 2)
        return carry

    lax.fori_loop(0, (_ANCH - 5) // 4, quad, 0)
    jj = _ANCH - 5
    finish(jj, jj % 4)
    start(jj + 4, (jj + 4) % 4)
    arm(jj + 3, (jj + 3) % 4)
    finish(jj + 1, (jj + 1) % 4)
    arm(jj + 4, (jj + 4) % 4)
    finish(jj + 2, (jj + 2) % 4)
    finish(jj + 3, (jj + 3) % 4)
    finish(jj + 4, (jj + 4) % 4)
    plsc.subcore_barrier()
    pltpu.sync_copy(acc.at[pl.ds(s * _RPT, _RPT)],
                    out_hbm.at[pl.ds(c * _NP + s * _RPT, _RPT)])


def _tc_first_body(deg0_ref, deg1_ref, x_ref, w_ref, dis_ref, hwp_ref):
    p0 = deg0_ref[:, 0:1]
    p1 = deg1_ref[:, 0:1]
    dis = lax.rsqrt(1.0 + p0 + p1)
    dis_b = jnp.broadcast_to(dis, (_BLK, _D))
    hw = jnp.dot(x_ref[...], w_ref[...], preferred_element_type=jnp.float32)
    dis_ref[...] = dis_b
    hwp_ref[...] = hw * dis_b


def _tc_first(deg, x, W1):
    return pl.pallas_call(
        _tc_first_body,
        grid=(_GRID,),
        in_specs=[
            pl.BlockSpec((_BLK, _D), lambda j: (j, 0)),
            pl.BlockSpec((_BLK, _D), lambda j: (j + _GRID, 0)),
            pl.BlockSpec((_BLK, _D), lambda j: (j, 0)),
            pl.BlockSpec((_D, _D), lambda j: (0, 0)),
        ],
        out_specs=[
            pl.BlockSpec((_BLK, _D), lambda j: (j, 0)),
            pl.BlockSpec((_BLK, _D), lambda j: (j, 0)),
        ],
        out_shape=[
            jax.ShapeDtypeStruct((_NP, _D), jnp.float32),
            jax.ShapeDtypeStruct((_NP, _D), jnp.float32),
        ],
    )(deg, deg, x, W1)


def _tc_mid_body(s0_ref, s1_ref, hwp_ref, dis_ref, b_ref, w_ref, out_ref):
    h = dis_ref[...] * (s0_ref[...] + s1_ref[...] + hwp_ref[...]) + b_ref[...]
    h = jnp.maximum(h, 0.0)
    out_ref[...] = (
        jnp.dot(h, w_ref[...], preferred_element_type=jnp.float32)
        * dis_ref[...]
    )


def _tc_mid(S, hwp, dis, b, W):
    return pl.pallas_call(
        _tc_mid_body,
        grid=(_GRID,),
        in_specs=[
            pl.BlockSpec((_BLK, _D), lambda j: (j, 0)),
            pl.BlockSpec((_BLK, _D), lambda j: (j + _GRID, 0)),
            pl.BlockSpec((_BLK, _D), lambda j: (j, 0)),
            pl.BlockSpec((_BLK, _D), lambda j: (j, 0)),
            pl.BlockSpec((1, _D), lambda j: (0, 0)),
            pl.BlockSpec((_D, _D), lambda j: (0, 0)),
        ],
        out_specs=pl.BlockSpec((_BLK, _D), lambda j: (j, 0)),
        out_shape=jax.ShapeDtypeStruct((_NP, _D), jnp.float32),
    )(S, S, hwp, dis, b, W)


def _tc_last_body(s0_ref, s1_ref, hwp_ref, dis_ref, b_ref, wo_ref, bo_ref,
                  out_ref):
    h = dis_ref[...] * (s0_ref[...] + s1_ref[...] + hwp_ref[...]) + b_ref[...]
    h = jnp.maximum(h, 0.0)
    z = jnp.dot(h, wo_ref[...], preferred_element_type=jnp.float32) + bo_ref[...]
    out_ref[...] = jax.nn.sigmoid(z)


def _tc_last(S, hwp, dis, b3, Wo, bo):
    return pl.pallas_call(
        _tc_last_body,
        grid=(_GRID,),
        in_specs=[
            pl.BlockSpec((_BLK, _D), lambda j: (j, 0)),
            pl.BlockSpec((_BLK, _D), lambda j: (j + _GRID, 0)),
            pl.BlockSpec((_BLK, _D), lambda j: (j, 0)),
            pl.BlockSpec((_BLK, _D), lambda j: (j, 0)),
            pl.BlockSpec((1, _D), lambda j: (0, 0)),
            pl.BlockSpec((_D, 1), lambda j: (0, 0)),
            pl.BlockSpec((1, 1), lambda j: (0, 0)),
        ],
        out_specs=pl.BlockSpec((_BLK, 1), lambda j: (j, 0)),
        out_shape=jax.ShapeDtypeStruct((_N, 1), jnp.float32),
    )(S, S, hwp, dis, b3, Wo, bo)


def kernel(x, edge_index, W1, b1, W2, b2, W3, b3, Wo, bo):
    src1 = edge_index[0]
    dst2 = edge_index[1].reshape(_NW, _NCH, _CH)
    zeros = jnp.zeros((_NP, _D), jnp.float32)
    ones = jnp.ones((_CH, _D), jnp.float32)

    deg = _sc_deg(dst2, zeros, ones)
    dis, hw1p = _tc_first(deg, x, W1)
    dst1 = edge_index[1]
    S1 = _sc_agg(hw1p, src1, dst1, zeros)
    hw2p = _tc_mid(S1, hw1p, dis, b1.reshape(1, _D), W2)
    S2 = _sc_agg(hw2p, src1, dst1, zeros)
    hw3p = _tc_mid(S2, hw2p, dis, b2.reshape(1, _D), W3)
    S3 = _sc_agg(hw3p, src1, dst1, zeros)
    return _tc_last(S3, hw3p, dis, b3.reshape(1, _D), Wo, bo.reshape(1, 1))
